# baseline (speedup 1.0000x reference)
import functools

import jax
import jax.numpy as jnp
from jax.experimental import pallas as pl
from jax.experimental.pallas import tpu as pltpu

_LOG_2PI = 1.8378770664093453
_L2E = 1.4426950408889634
_LN2 = 0.6931471805599453
_LOG2_2PI = _LOG_2PI * _L2E


def _gmm_mxu_kernel(a_ref, ch_ref, g_ref, bgt_ref, mask_ref, res_ref, *, nc):
    ch = ch_ref[0]
    g = g_ref[...]

    p = ch[0:1, :]
    bg = ch[9:10, :]
    d_bg = bg - bgt_ref[0]
    psum = jnp.sum(p, axis=(0, 1), keepdims=True)
    pvar = jnp.sum(p - p * p, axis=(0, 1), keepdims=True)
    sqerr = jnp.sum(d_bg * d_bg, axis=(0, 1), keepdims=True)

    acols = a_ref[0]
    ahi = acols.astype(jnp.bfloat16)
    arem = acols - ahi.astype(jnp.float32)
    amid = arem.astype(jnp.bfloat16)
    alo = (arem - amid.astype(jnp.float32)).astype(jnp.bfloat16)
    amat = jnp.concatenate([ahi, ahi, amid, ahi, alo, amid], axis=1)

    K = ch.shape[1]
    chunk = K // nc
    ms = []
    ls = []
    for c in range(nc):
        sl = slice(c * chunk, (c + 1) * chunk)
        sigs = ch[5:8, sl]
        mus = jnp.concatenate(
            [ch[2:4, sl] + g[:, sl], ch[4:5, sl] + 0.01], axis=0)
        rcp = 1.0 / sigs
        q = (0.5 * _L2E) * rcp * rcp
        qmu = q * mus
        logs = jnp.log2(jnp.concatenate(
            [jnp.maximum(p[:, sl], 1e-37), sigs], axis=0))
        b0 = (2.0 * logs[0:1, :] - jnp.sum(logs, axis=0, keepdims=True)
              - 1.5 * _LOG2_2PI - jnp.sum(qmu * mus, axis=0, keepdims=True))
        rows = jnp.concatenate(
            [b0, 2.0 * qmu, -q, jnp.zeros_like(b0)], axis=0)

        hi = rows.astype(jnp.bfloat16)
        rem = rows - hi.astype(jnp.float32)
        mid = rem.astype(jnp.bfloat16)
        lo = (rem - mid.astype(jnp.float32)).astype(jnp.bfloat16)
        bmat = jnp.concatenate([hi, mid, hi, lo, hi, mid], axis=0)

        lp2 = jax.lax.dot_general(
            amat, bmat, (((1,), (0,)), ((), ())),
            preferred_element_type=jnp.float32)
        m_c = jnp.max(lp2, axis=-1, keepdims=True)
        l_c = jnp.sum(jnp.exp2(lp2 - m_c), axis=-1, keepdims=True)
        ms.append(m_c)
        ls.append(l_c)

    if nc > 1:
        mall = jnp.concatenate(ms, axis=1)
        lall = jnp.concatenate(ls, axis=1)
        m = jnp.max(mall, axis=-1, keepdims=True)
        l = jnp.sum(jnp.exp2(mall - m) * lall, axis=-1, keepdims=True)
    else:
        m, l = ms[0], ls[0]

    lse = _LN2 * (m + jnp.log2(l)) - jnp.log(psum)
    gmm = jnp.sum(-lse * mask_ref[0], axis=(0, 1), keepdims=True)
    res_ref[0] = jnp.concatenate([gmm, psum, pvar, sqerr], axis=-1)


def _pick_chunks(K):
    for chunk in (4096, 2048, 1024, 512, 256, 128):
        if K % chunk == 0:
            return K // chunk
    return 1


def _stats_call(acols, out_flat, grid_c, bgt, mask3):
    B, C, K = out_flat.shape
    M = acols.shape[1]
    nc = _pick_chunks(K)

    cost = pl.CostEstimate(
        flops=int(B * K * (2 * 48 * M + 48)),
        transcendentals=int(B * K * (M + 8)),
        bytes_accessed=int(4 * B * ((C + 1) * K + 9 * M + 4) + 8 * K),
    )

    kern = functools.partial(_gmm_mxu_kernel, nc=nc)
    return pl.pallas_call(
        kern,
        out_shape=jax.ShapeDtypeStruct((B, 1, 4), jnp.float32),
        grid_spec=pltpu.PrefetchScalarGridSpec(
            num_scalar_prefetch=0,
            grid=(B,),
            in_specs=[
                pl.BlockSpec((1, M, 8), lambda b: (b, 0, 0)),
                pl.BlockSpec((1, C, K), lambda b: (b, 0, 0)),
                pl.BlockSpec((2, K), lambda b: (0, 0)),
                pl.BlockSpec((1, 1, K), lambda b: (b, 0, 0)),
                pl.BlockSpec((1, M, 1), lambda b: (b, 0, 0)),
            ],
            out_specs=pl.BlockSpec((1, 1, 4), lambda b: (b, 0, 0)),
            scratch_shapes=[]),
        compiler_params=pltpu.CompilerParams(
            dimension_semantics=("arbitrary",),
            vmem_limit_bytes=100 << 20),
        cost_estimate=cost,
    )(acols, out_flat, grid_c, bgt, mask3)


def kernel(output, pos, mask, bg_truth):
    B, C, H, W = output.shape
    K = H * W

    out_flat = output.reshape(B, C, K).astype(jnp.float32)
    bgt = bg_truth.reshape(B, 1, K).astype(jnp.float32)
    truth = pos.reshape(B, -1, 3).astype(jnp.float32)
    truth = truth.at[:, :, 2].set(truth[:, :, 2] / jnp.max(truth[:, :, 2]))
    M = truth.shape[1]
    mask3 = mask.reshape(B, M, 1).astype(jnp.float32)

    cy, cx = 0.5 * H, 0.5 * W
    gy, gx = jnp.meshgrid(jnp.arange(H, dtype=jnp.float32) + 0.5 - cy,
                          jnp.arange(W, dtype=jnp.float32) + 0.5 - cx,
                          indexing="ij")
    grid_c = jnp.stack([gy, gx], 0).reshape(2, K)

    t0 = truth[:, :, 0] - cy
    t1 = truth[:, :, 1] - cx
    t2 = truth[:, :, 2]
    acols = jnp.stack([jnp.ones_like(t0), t0, t1, t2, t0 * t0, t1 * t1,
                       t2 * t2, jnp.zeros_like(t0)], axis=-1)

    stats = _stats_call(acols, out_flat, grid_c, bgt, mask3)

    gmm_loss = jnp.sum(stats[:, 0, 0])
    p_mean = stats[:, 0, 1]
    p_var = stats[:, 0, 2]
    sqerr = stats[:, 0, 3]

    n = jnp.sum(mask, axis=-1)
    log_prob_n = (-0.5 * (n - p_mean) ** 2 / p_var
                  - 0.5 * jnp.log(p_var) - 0.5 * _LOG_2PI)
    c_loss = jnp.sum(-log_prob_n * n) / 10.0

    bg_loss = jnp.sum(sqerr) / (B * K) * 10.0

    return gmm_loss + c_loss + bg_loss


# speedup vs baseline: 1.0015x; 1.0015x over previous
import functools

import jax
import jax.numpy as jnp
from jax.experimental import pallas as pl
from jax.experimental.pallas import tpu as pltpu

_LOG_2PI = 1.8378770664093453
_L2E = 1.4426950408889634
_LN2 = 0.6931471805599453
_LOG2_2PI = _LOG_2PI * _L2E


def _gmm_mxu_kernel(a_ref, ch_ref, g_ref, bgt_ref, mask_ref, res_ref, *, nc):
    ch = ch_ref[0]
    g = g_ref[...]

    p = ch[0:1, :]
    bg = ch[9:10, :]
    d_bg = bg - bgt_ref[0]
    psum = jnp.sum(p, axis=(0, 1), keepdims=True)
    pvar = jnp.sum(p - p * p, axis=(0, 1), keepdims=True)
    sqerr = jnp.sum(d_bg * d_bg, axis=(0, 1), keepdims=True)

    acols = a_ref[0]
    ahi = acols.astype(jnp.bfloat16)
    arem = acols - ahi.astype(jnp.float32)
    amid = arem.astype(jnp.bfloat16)
    alo = (arem - amid.astype(jnp.float32)).astype(jnp.bfloat16)
    amat = jnp.concatenate([ahi, ahi, amid, ahi, alo, amid], axis=1)

    K = ch.shape[1]
    chunk = K // nc
    ms = []
    ls = []
    for c in range(nc):
        sl = slice(c * chunk, (c + 1) * chunk)
        sigs = ch[5:8, sl]
        mus = jnp.concatenate(
            [ch[2:4, sl] + g[:, sl], ch[4:5, sl] + 0.01], axis=0)
        rcp = 1.0 / sigs
        q = (0.5 * _L2E) * rcp * rcp
        qmu = q * mus
        logs = jnp.log2(jnp.concatenate(
            [jnp.maximum(p[:, sl], 1e-37), sigs], axis=0))
        b0 = (2.0 * logs[0:1, :] - jnp.sum(logs, axis=0, keepdims=True)
              - 1.5 * _LOG2_2PI - jnp.sum(qmu * mus, axis=0, keepdims=True))
        rows = jnp.concatenate(
            [b0, 2.0 * qmu, -q, jnp.zeros_like(b0)], axis=0)

        hi = rows.astype(jnp.bfloat16)
        rem = rows - hi.astype(jnp.float32)
        mid = rem.astype(jnp.bfloat16)
        lo = (rem - mid.astype(jnp.float32)).astype(jnp.bfloat16)
        bmat = jnp.concatenate([hi, mid, hi, lo, hi, mid], axis=0)

        lp2 = jax.lax.dot_general(
            amat, bmat, (((1,), (0,)), ((), ())),
            preferred_element_type=jnp.float32)
        m_c = jnp.max(lp2, axis=-1, keepdims=True)
        l_c = jnp.sum(jnp.exp2(lp2 - m_c), axis=-1, keepdims=True)
        ms.append(m_c)
        ls.append(l_c)

    if nc > 1:
        mall = jnp.concatenate(ms, axis=1)
        lall = jnp.concatenate(ls, axis=1)
        m = jnp.max(mall, axis=-1, keepdims=True)
        l = jnp.sum(jnp.exp2(mall - m) * lall, axis=-1, keepdims=True)
    else:
        m, l = ms[0], ls[0]

    lse = _LN2 * (m + jnp.log2(l)) - jnp.log(psum)
    gmm = jnp.sum(-lse * mask_ref[0], axis=(0, 1), keepdims=True)
    res_ref[0] = jnp.concatenate([gmm, psum, pvar, sqerr], axis=-1)


def _pick_chunks(K):
    for chunk in (2048, 1024, 512, 256, 128):
        if K % chunk == 0:
            return K // chunk
    return 1


def _stats_call(acols, out_flat, grid_c, bgt, mask3):
    B, C, K = out_flat.shape
    M = acols.shape[1]
    nc = _pick_chunks(K)

    cost = pl.CostEstimate(
        flops=int(B * K * (2 * 48 * M + 48)),
        transcendentals=int(B * K * (M + 8)),
        bytes_accessed=int(4 * B * ((C + 1) * K + 9 * M + 4) + 8 * K),
    )

    kern = functools.partial(_gmm_mxu_kernel, nc=nc)
    return pl.pallas_call(
        kern,
        out_shape=jax.ShapeDtypeStruct((B, 1, 4), jnp.float32),
        grid_spec=pltpu.PrefetchScalarGridSpec(
            num_scalar_prefetch=0,
            grid=(B,),
            in_specs=[
                pl.BlockSpec((1, M, 8), lambda b: (b, 0, 0)),
                pl.BlockSpec((1, C, K), lambda b: (b, 0, 0)),
                pl.BlockSpec((2, K), lambda b: (0, 0)),
                pl.BlockSpec((1, 1, K), lambda b: (b, 0, 0)),
                pl.BlockSpec((1, M, 1), lambda b: (b, 0, 0)),
            ],
            out_specs=pl.BlockSpec((1, 1, 4), lambda b: (b, 0, 0)),
            scratch_shapes=[]),
        compiler_params=pltpu.CompilerParams(
            dimension_semantics=("arbitrary",),
            vmem_limit_bytes=100 << 20),
        cost_estimate=cost,
    )(acols, out_flat, grid_c, bgt, mask3)


def kernel(output, pos, mask, bg_truth):
    B, C, H, W = output.shape
    K = H * W

    out_flat = output.reshape(B, C, K).astype(jnp.float32)
    bgt = bg_truth.reshape(B, 1, K).astype(jnp.float32)
    truth = pos.reshape(B, -1, 3).astype(jnp.float32)
    truth = truth.at[:, :, 2].set(truth[:, :, 2] / jnp.max(truth[:, :, 2]))
    M = truth.shape[1]
    mask3 = mask.reshape(B, M, 1).astype(jnp.float32)

    cy, cx = 0.5 * H, 0.5 * W
    gy, gx = jnp.meshgrid(jnp.arange(H, dtype=jnp.float32) + 0.5 - cy,
                          jnp.arange(W, dtype=jnp.float32) + 0.5 - cx,
                          indexing="ij")
    grid_c = jnp.stack([gy, gx], 0).reshape(2, K)

    t0 = truth[:, :, 0] - cy
    t1 = truth[:, :, 1] - cx
    t2 = truth[:, :, 2]
    acols = jnp.stack([jnp.ones_like(t0), t0, t1, t2, t0 * t0, t1 * t1,
                       t2 * t2, jnp.zeros_like(t0)], axis=-1)

    stats = _stats_call(acols, out_flat, grid_c, bgt, mask3)

    gmm_loss = jnp.sum(stats[:, 0, 0])
    p_mean = stats[:, 0, 1]
    p_var = stats[:, 0, 2]
    sqerr = stats[:, 0, 3]

    n = jnp.sum(mask, axis=-1)
    log_prob_n = (-0.5 * (n - p_mean) ** 2 / p_var
                  - 0.5 * jnp.log(p_var) - 0.5 * _LOG_2PI)
    c_loss = jnp.sum(-log_prob_n * n) / 10.0

    bg_loss = jnp.sum(sqerr) / (B * K) * 10.0

    return gmm_loss + c_loss + bg_loss
